# Initial kernel scaffold; baseline (speedup 1.0000x reference)
import jax
import jax.numpy as jnp
from jax.experimental import pallas as pl


def kernel(t, embed_table):
    raise NotImplementedError("write your pallas kernel here")


# speedup vs baseline: 1.8288x; 1.8288x over previous
import functools

import jax
import jax.numpy as jnp
import numpy as np
from jax import lax
from jax.experimental import pallas as pl
from jax.experimental.pallas import tpu as pltpu
from jax.experimental.pallas import tpu_sc as plsc

MAXF = 1024
D_EMB = 256
NFEAT = 10
D_OUT = D_EMB + 2 * NFEAT
D_PAD = 288
B = 16384
NDIFF = 2
NROWS = B * NDIFF

NC, NS = 2, 16
NW = NC * NS
RW = NROWS // NW
CHUNK = 64
NCHUNK = RW // CHUNK
CWORDS = CHUNK * D_OUT
GV = CWORDS // 16
MAGIC, SHIFT = 15197, 22


def _diff_body(t_ref, d_ref):
    t = t_ref[...]
    d_ref[...] = t[:, 1:3] - t[:, 0:2]


def _table_body(emb_ref, tab_ref):
    d = lax.broadcasted_iota(jnp.int32, (MAXF, NFEAT), 0).astype(jnp.float32)
    j = lax.broadcasted_iota(jnp.int32, (MAXF, NFEAT), 1).astype(jnp.float32)
    raw = d * jnp.exp2(j) * (np.pi / MAXF)
    pad = jnp.zeros((MAXF, D_PAD - D_OUT), jnp.float32)
    tab_ref[...] = jnp.concatenate(
        [emb_ref[...], jnp.sin(raw), jnp.cos(raw), pad], axis=1)


def _sc_body(tab_hbm, idx_hbm, out_hbm,
             idx_v, g0, g1, c0, c1, gs0, gs1, ss0, ss1):
    wid = lax.axis_index("s") * NC + lax.axis_index("c")
    pltpu.sync_copy(idx_hbm.at[pl.ds(wid * NCHUNK, NCHUNK)], idx_v)
    gbufs, cbufs = (g0, g1), (c0, c1)
    gsems, ssems = (gs0, gs1), (ss0, ss1)
    out_base = pl.multiple_of(wid * (RW * D_OUT), RW * D_OUT)

    def start_gather(c):
        return pltpu.async_copy(tab_hbm.at[idx_v.at[c]], gbufs[c % 2],
                                gsems[c % 2])

    g = {0: start_gather(0), 1: start_gather(1)}
    s = {}
    for c in range(NCHUNK):
        if c >= 2:
            s[c - 2].wait()
        g[c].wait()
        gbuf, cbuf = gbufs[c % 2], cbufs[c % 2]

        @plsc.parallel_loop(0, GV, 1, unroll=8)
        def _compact(v):
            w = v * 16 + lax.iota(jnp.int32, 16)
            r = (w * MAGIC) >> SHIFT
            col = w - r * D_OUT
            vals = plsc.load_gather(gbuf, [r, col])
            cbuf[pl.ds(pl.multiple_of(v * 16, 16), 16)] = vals

        s[c] = pltpu.async_copy(
            cbuf, out_hbm.at[pl.ds(out_base + c * CWORDS, CWORDS)],
            ssems[c % 2])
        if c + 2 < NCHUNK:
            g[c + 2] = start_gather(c + 2)
    s[NCHUNK - 2].wait()
    s[NCHUNK - 1].wait()


def kernel(t, embed_table):
    diffs = pl.pallas_call(
        _diff_body,
        grid=(16,),
        in_specs=[pl.BlockSpec((B // 16, 3), lambda i: (i, 0))],
        out_specs=pl.BlockSpec((B // 16, NDIFF), lambda i: (i, 0)),
        out_shape=jax.ShapeDtypeStruct((B, NDIFF), jnp.int32),
    )(t)

    fused_tab = pl.pallas_call(
        _table_body,
        out_shape=jax.ShapeDtypeStruct((MAXF, D_PAD), jnp.float32),
    )(embed_table)

    idx2d = diffs.reshape(NROWS // CHUNK, CHUNK)

    mesh = plsc.VectorSubcoreMesh(core_axis_name="c", subcore_axis_name="s")
    sc_gather = functools.partial(
        pl.kernel,
        mesh=mesh,
        compiler_params=pltpu.CompilerParams(
            use_tc_tiling_on_sc=False, needs_layout_passes=False),
        out_type=jax.ShapeDtypeStruct((NROWS * D_OUT,), jnp.float32),
        scratch_types=[
            pltpu.VMEM((NCHUNK, CHUNK), jnp.int32),
            pltpu.VMEM((CHUNK, D_PAD), jnp.float32),
            pltpu.VMEM((CHUNK, D_PAD), jnp.float32),
            pltpu.VMEM((CWORDS,), jnp.float32),
            pltpu.VMEM((CWORDS,), jnp.float32),
            pltpu.SemaphoreType.DMA,
            pltpu.SemaphoreType.DMA,
            pltpu.SemaphoreType.DMA,
            pltpu.SemaphoreType.DMA,
        ],
    )(_sc_body)

    out = sc_gather(fused_tab, idx2d)
    return out.reshape(B, NDIFF * D_OUT)
